# Initial kernel scaffold; baseline (speedup 1.0000x reference)
import jax
import jax.numpy as jnp
from jax.experimental import pallas as pl


def kernel(z, pos, batch, edge_index, edge_weight, emb, mlp_w1, mlp_b1, mlp_w2, mlp_b2, conv_w1, conv_w2, conv_b2, int_w, int_b, lin1_w, lin1_b, lin2_w, lin2_b):
    raise NotImplementedError("write your pallas kernel here")


# speedup vs baseline: 3.3570x; 3.3570x over previous
import functools

import jax
import jax.numpy as jnp
from jax import lax
from jax.experimental import pallas as pl
from jax.experimental.pallas import tpu as pltpu
from jax.experimental.pallas import tpu_sc as plsc

N = 10000
NPAD = 10240
E = 320000
H = 128
F = 128
G = 50
L = 6
CUT = 10.0

NC = 2
NS = 16
NW = NC * NS
EPW = E // NW
CH = 128
NFULL = EPW // CH
TAIL = EPW - NFULL * CH
RPT = NPAD // NS

EPAD = 327680
RT_F = 4096
RT_N = 1280
NBLK = NPAD // RT_N

_LOG2 = 0.6931471805599453


def _ssp(x):
    return jnp.maximum(x, 0.0) + jnp.log(1.0 + jnp.exp(-jnp.abs(x))) - _LOG2


def _filter_body(ew_ref, w1_ref, b1_ref, w2_ref, b2_ref, out_ref):
    ew = ew_ref[:]
    step = CUT / (G - 1)
    off = lax.broadcasted_iota(jnp.int32, (RT_F, G), 1).astype(jnp.float32) * step
    coeff = -0.5 / step ** 2
    a = jnp.exp(coeff * (ew[:, None] - off) ** 2)
    t = _ssp(jnp.dot(a, w1_ref[:], preferred_element_type=jnp.float32) + b1_ref[:])
    w = jnp.dot(t, w2_ref[:], preferred_element_type=jnp.float32) + b2_ref[:]
    c = 0.5 * (jnp.cos(ew * (jnp.pi / CUT)) + 1.0)
    out_ref[:] = w * c[:, None]


def _filter_call(ew, w1, b1, w2, b2):
    return pl.pallas_call(
        _filter_body,
        grid=(EPAD // RT_F,),
        in_specs=[
            pl.BlockSpec((RT_F,), lambda i: (i,)),
            pl.BlockSpec((G, F), lambda i: (0, 0)),
            pl.BlockSpec((F,), lambda i: (0,)),
            pl.BlockSpec((F, F), lambda i: (0, 0)),
            pl.BlockSpec((F,), lambda i: (0,)),
        ],
        out_specs=pl.BlockSpec((RT_F, F), lambda i: (i, 0)),
        out_shape=jax.ShapeDtypeStruct((EPAD, F), jnp.float32),
    )(ew, w1, b1, w2, b2)


def _embed_body(z_ref, emb_ref, w1_ref, h_ref, xc_ref):
    z = z_ref[0, 0].reshape(RT_N, 1)
    oh = (lax.broadcasted_iota(jnp.int32, (RT_N, 120), 1) == z).astype(jnp.float32)
    h = jnp.dot(oh, emb_ref[:], preferred_element_type=jnp.float32)
    h_ref[:] = h
    xc_ref[:] = jnp.dot(h, w1_ref[:], preferred_element_type=jnp.float32)


def _embed_call(zp, emb, w1):
    return pl.pallas_call(
        _embed_body,
        grid=(NBLK,),
        in_specs=[
            pl.BlockSpec((1, 1, RT_N), lambda i: (i, 0, 0)),
            pl.BlockSpec((120, H), lambda i: (0, 0)),
            pl.BlockSpec((H, F), lambda i: (0, 0)),
        ],
        out_specs=[
            pl.BlockSpec((RT_N, H), lambda i: (i, 0)),
            pl.BlockSpec((RT_N, F), lambda i: (i, 0)),
        ],
        out_shape=[
            jax.ShapeDtypeStruct((NPAD, H), jnp.float32),
            jax.ShapeDtypeStruct((NPAD, F), jnp.float32),
        ],
    )(zp, emb, w1)


def _node_body(p0_ref, p1_ref, h_ref, w2_ref, b2_ref, iw_ref, ib_ref, w1n_ref,
               hn_ref, xcn_ref):
    agg = p0_ref[:] + p1_ref[:]
    x2 = jnp.dot(agg, w2_ref[:], preferred_element_type=jnp.float32) + b2_ref[:]
    hi = jnp.dot(_ssp(x2), iw_ref[:], preferred_element_type=jnp.float32) + ib_ref[:]
    hn = h_ref[:] + hi
    hn_ref[:] = hn
    xcn_ref[:] = jnp.dot(hn, w1n_ref[:], preferred_element_type=jnp.float32)


def _node_call(parts, h, w2, b2, iw, ib, w1n):
    return pl.pallas_call(
        _node_body,
        grid=(NBLK,),
        in_specs=[
            pl.BlockSpec((RT_N, H), lambda i: (i, 0)),
            pl.BlockSpec((RT_N, H), lambda i: (i + NBLK, 0)),
            pl.BlockSpec((RT_N, H), lambda i: (i, 0)),
            pl.BlockSpec((F, H), lambda i: (0, 0)),
            pl.BlockSpec((H,), lambda i: (0,)),
            pl.BlockSpec((H, H), lambda i: (0, 0)),
            pl.BlockSpec((H,), lambda i: (0,)),
            pl.BlockSpec((H, F), lambda i: (0, 0)),
        ],
        out_specs=[
            pl.BlockSpec((RT_N, H), lambda i: (i, 0)),
            pl.BlockSpec((RT_N, F), lambda i: (i, 0)),
        ],
        out_shape=[
            jax.ShapeDtypeStruct((NPAD, H), jnp.float32),
            jax.ShapeDtypeStruct((NPAD, F), jnp.float32),
        ],
    )(parts, parts, h, w2, b2, iw, ib, w1n)


def _final_body(h_ref, w1_ref, b1_ref, w2_ref, b2_ref, out_ref):
    i = pl.program_id(0)
    t = _ssp(jnp.dot(h_ref[:], w1_ref[:], preferred_element_type=jnp.float32) + b1_ref[:])
    r = jnp.sum(t * w2_ref[:], axis=1, keepdims=True) + b2_ref[:]
    rows = lax.broadcasted_iota(jnp.int32, (RT_N, 1), 0) + i * RT_N
    s = jnp.sum(jnp.where(rows < N, r, 0.0)).reshape(1, 1)

    @pl.when(i == 0)
    def _():
        out_ref[:] = s

    @pl.when(i != 0)
    def _():
        out_ref[:] = out_ref[:] + s


def _final_call(h, w1, b1, w2v, b2):
    return pl.pallas_call(
        _final_body,
        grid=(NBLK,),
        in_specs=[
            pl.BlockSpec((RT_N, H), lambda i: (i, 0)),
            pl.BlockSpec((H, H // 2), lambda i: (0, 0)),
            pl.BlockSpec((H // 2,), lambda i: (0,)),
            pl.BlockSpec((1, H // 2), lambda i: (0, 0)),
            pl.BlockSpec((1,), lambda i: (0,)),
        ],
        out_specs=pl.BlockSpec((1, 1), lambda i: (0, 0)),
        out_shape=jax.ShapeDtypeStruct((1, 1), jnp.float32),
    )(h, w1, b1, w2v, b2)


def _sc_body(xc_hbm, w_hbm, src_hbm, dst_hbm, out_hbm,
             acc, sidx, didx, rows, wrow, sidx_t, didx_t, rows_t, wrow_t):
    c = lax.axis_index("c")
    s = lax.axis_index("s")
    wid = c * NS + s

    @pl.loop(0, CH)
    def _(r):
        for cb in range(H // 16):
            rows[r, pl.ds(cb * 16, 16)] = jnp.zeros((16,), jnp.float32)

    rbase = s * RPT
    for q in range(RPT // CH):
        pltpu.sync_copy(rows, acc.at[pl.ds(rbase + q * CH, CH)])
    plsc.subcore_barrier()

    ebase = wid * EPW

    def do_chunk(off, si, di, rv, wv, k):
        pltpu.sync_copy(src_hbm.at[pl.ds(off, k)], si)
        pltpu.sync_copy(dst_hbm.at[pl.ds(off, k)], di)
        pltpu.sync_copy(xc_hbm.at[si], rv)
        pltpu.sync_copy(w_hbm.at[pl.ds(off, k)], wv)

        @pl.loop(0, k)
        def _(r):
            for cb in range(H // 16):
                sl = pl.ds(cb * 16, 16)
                rv[r, sl] = rv[r, sl] * wv[r, sl]

        pltpu.sync_copy(rv, acc.at[di], add=True)

    @pl.loop(0, NFULL)
    def _(j):
        do_chunk(pl.multiple_of(ebase + j * CH, 8), sidx, didx, rows, wrow, CH)

    do_chunk(pl.multiple_of(ebase + NFULL * CH, 8),
             sidx_t, didx_t, rows_t, wrow_t, TAIL)

    plsc.subcore_barrier()
    pltpu.sync_copy(acc.at[pl.ds(rbase, RPT)],
                    out_hbm.at[pl.ds(c * NPAD + rbase, RPT)])


@functools.lru_cache(maxsize=1)
def _sc_gms_built():
    mesh = plsc.VectorSubcoreMesh(core_axis_name="c", subcore_axis_name="s")
    return pl.kernel(
        _sc_body,
        mesh=mesh,
        out_type=jax.ShapeDtypeStruct((NC * NPAD, H), jnp.float32),
        scratch_types=[
            pltpu.VMEM_SHARED((NPAD, H), jnp.float32),
            pltpu.VMEM((CH,), jnp.int32),
            pltpu.VMEM((CH,), jnp.int32),
            pltpu.VMEM((CH, H), jnp.float32),
            pltpu.VMEM((CH, H), jnp.float32),
            pltpu.VMEM((TAIL,), jnp.int32),
            pltpu.VMEM((TAIL,), jnp.int32),
            pltpu.VMEM((TAIL, H), jnp.float32),
            pltpu.VMEM((TAIL, H), jnp.float32),
        ],
    )


def _sc_call(xc, w, src, dst):
    return _sc_gms_built()(xc, w, src, dst)


def kernel(z, pos, batch, edge_index, edge_weight, emb,
           mlp_w1, mlp_b1, mlp_w2, mlp_b2,
           conv_w1, conv_w2, conv_b2, int_w, int_b,
           lin1_w, lin1_b, lin2_w, lin2_b):
    src = edge_index[0].astype(jnp.int32)
    dst = edge_index[1].astype(jnp.int32)
    zp = jnp.pad(z.astype(jnp.int32), (0, NPAD - N)).reshape(NBLK, 1, RT_N)
    ewp = jnp.pad(edge_weight, (0, EPAD - E))

    ws = [_filter_call(ewp, mlp_w1[i], mlp_b1[i], mlp_w2[i], mlp_b2[i])
          for i in range(L)]

    h, xc = _embed_call(zp, emb, conv_w1[0])
    for i in range(L):
        parts = _sc_call(xc, ws[i], src, dst)
        w1n = conv_w1[i + 1] if i + 1 < L else conv_w1[0]
        h, xc = _node_call(parts, h, conv_w2[i], conv_b2[i], int_w[i], int_b[i], w1n)

    return _final_call(h, lin1_w, lin1_b, lin2_w.reshape(1, H // 2), lin2_b)


# speedup vs baseline: 4.2840x; 1.2761x over previous
import functools

import jax
import jax.numpy as jnp
from jax import lax
from jax.experimental import pallas as pl
from jax.experimental.pallas import tpu as pltpu
from jax.experimental.pallas import tpu_sc as plsc

N = 10000
NPAD = 10240
E = 320000
H = 128
F = 128
G = 50
L = 6
CUT = 10.0

NC = 2
NS = 16
NW = NC * NS
EPW = E // NW
CH = 64
NF2 = 158
EPP = NF2 * CH
EPAD2 = NW * EPP
RPT = NPAD // NS

RT_F = 4096
RT_N = 1280
NBLK = NPAD // RT_N

_LOG2 = 0.6931471805599453


def _ssp(x):
    return jnp.maximum(x, 0.0) + jnp.log(1.0 + jnp.exp(-jnp.abs(x))) - _LOG2


def _filter_body(ew_ref, w1_ref, b1_ref, w2_ref, b2_ref, out_ref):
    ew = ew_ref[:]
    step = CUT / (G - 1)
    off = lax.broadcasted_iota(jnp.int32, (RT_F, G), 1).astype(jnp.float32) * step
    coeff = -0.5 / step ** 2
    a = jnp.exp(coeff * (ew[:, None] - off) ** 2)
    t = _ssp(jnp.dot(a, w1_ref[:], preferred_element_type=jnp.float32) + b1_ref[:])
    w = jnp.dot(t, w2_ref[:], preferred_element_type=jnp.float32) + b2_ref[:]
    c = 0.5 * (jnp.cos(ew * (jnp.pi / CUT)) + 1.0)
    out_ref[:] = w * c[:, None]


def _filter_call(ew, w1, b1, w2, b2):
    return pl.pallas_call(
        _filter_body,
        grid=(EPAD2 // RT_F,),
        in_specs=[
            pl.BlockSpec((RT_F,), lambda i: (i,)),
            pl.BlockSpec((G, F), lambda i: (0, 0)),
            pl.BlockSpec((F,), lambda i: (0,)),
            pl.BlockSpec((F, F), lambda i: (0, 0)),
            pl.BlockSpec((F,), lambda i: (0,)),
        ],
        out_specs=pl.BlockSpec((RT_F, F), lambda i: (i, 0)),
        out_shape=jax.ShapeDtypeStruct((EPAD2, F), jnp.float32),
    )(ew, w1, b1, w2, b2)


def _embed_body(z_ref, emb_ref, w1_ref, h_ref, xc_ref):
    z = z_ref[0, 0].reshape(RT_N, 1)
    oh = (lax.broadcasted_iota(jnp.int32, (RT_N, 120), 1) == z).astype(jnp.float32)
    h = jnp.dot(oh, emb_ref[:], preferred_element_type=jnp.float32)
    h_ref[:] = h
    xc_ref[:] = jnp.dot(h, w1_ref[:], preferred_element_type=jnp.float32)


def _embed_call(zp, emb, w1):
    return pl.pallas_call(
        _embed_body,
        grid=(NBLK,),
        in_specs=[
            pl.BlockSpec((1, 1, RT_N), lambda i: (i, 0, 0)),
            pl.BlockSpec((120, H), lambda i: (0, 0)),
            pl.BlockSpec((H, F), lambda i: (0, 0)),
        ],
        out_specs=[
            pl.BlockSpec((RT_N, H), lambda i: (i, 0)),
            pl.BlockSpec((RT_N, F), lambda i: (i, 0)),
        ],
        out_shape=[
            jax.ShapeDtypeStruct((NPAD, H), jnp.float32),
            jax.ShapeDtypeStruct((NPAD, F), jnp.float32),
        ],
    )(zp, emb, w1)


def _node_body(p0_ref, p1_ref, h_ref, w2_ref, b2_ref, iw_ref, ib_ref, w1n_ref,
               hn_ref, xcn_ref):
    agg = p0_ref[:] + p1_ref[:]
    x2 = jnp.dot(agg, w2_ref[:], preferred_element_type=jnp.float32) + b2_ref[:]
    hi = jnp.dot(_ssp(x2), iw_ref[:], preferred_element_type=jnp.float32) + ib_ref[:]
    hn = h_ref[:] + hi
    hn_ref[:] = hn
    xcn_ref[:] = jnp.dot(hn, w1n_ref[:], preferred_element_type=jnp.float32)


def _node_call(parts, h, w2, b2, iw, ib, w1n):
    return pl.pallas_call(
        _node_body,
        grid=(NBLK,),
        in_specs=[
            pl.BlockSpec((RT_N, H), lambda i: (i, 0)),
            pl.BlockSpec((RT_N, H), lambda i: (i + NBLK, 0)),
            pl.BlockSpec((RT_N, H), lambda i: (i, 0)),
            pl.BlockSpec((F, H), lambda i: (0, 0)),
            pl.BlockSpec((H,), lambda i: (0,)),
            pl.BlockSpec((H, H), lambda i: (0, 0)),
            pl.BlockSpec((H,), lambda i: (0,)),
            pl.BlockSpec((H, F), lambda i: (0, 0)),
        ],
        out_specs=[
            pl.BlockSpec((RT_N, H), lambda i: (i, 0)),
            pl.BlockSpec((RT_N, F), lambda i: (i, 0)),
        ],
        out_shape=[
            jax.ShapeDtypeStruct((NPAD, H), jnp.float32),
            jax.ShapeDtypeStruct((NPAD, F), jnp.float32),
        ],
    )(parts, parts, h, w2, b2, iw, ib, w1n)


def _final_body(h_ref, w1_ref, b1_ref, w2_ref, b2_ref, out_ref):
    i = pl.program_id(0)
    t = _ssp(jnp.dot(h_ref[:], w1_ref[:], preferred_element_type=jnp.float32) + b1_ref[:])
    r = jnp.sum(t * w2_ref[:], axis=1, keepdims=True) + b2_ref[:]
    rows = lax.broadcasted_iota(jnp.int32, (RT_N, 1), 0) + i * RT_N
    s = jnp.sum(jnp.where(rows < N, r, 0.0)).reshape(1, 1)

    @pl.when(i == 0)
    def _():
        out_ref[:] = s

    @pl.when(i != 0)
    def _():
        out_ref[:] = out_ref[:] + s


def _final_call(h, w1, b1, w2v, b2):
    return pl.pallas_call(
        _final_body,
        grid=(NBLK,),
        in_specs=[
            pl.BlockSpec((RT_N, H), lambda i: (i, 0)),
            pl.BlockSpec((H, H // 2), lambda i: (0, 0)),
            pl.BlockSpec((H // 2,), lambda i: (0,)),
            pl.BlockSpec((1, H // 2), lambda i: (0, 0)),
            pl.BlockSpec((1,), lambda i: (0,)),
        ],
        out_specs=pl.BlockSpec((1, 1), lambda i: (0, 0)),
        out_shape=jax.ShapeDtypeStruct((1, 1), jnp.float32),
    )(h, w1, b1, w2v, b2)


def _sc_body(xc_hbm, w_hbm, sp_hbm, dp_hbm, out_hbm,
             acc,
             rv0, rv1, wv0, wv1,
             is0, is1, is2, is3, id0, id1, id2, id3,
             gs0, gs1, ws0, ws1, ss0, ss1, es0, es1, es2, es3):
    c = lax.axis_index("c")
    s = lax.axis_index("s")
    wid = c * NS + s
    RV = (rv0, rv1)
    WV = (wv0, wv1)
    ISRC = (is0, is1, is2, is3)
    IDST = (id0, id1, id2, id3)
    GS = (gs0, gs1)
    WS = (ws0, ws1)
    SS = (ss0, ss1)
    ES = (es0, es1, es2, es3)

    @pl.loop(0, CH)
    def _(r):
        for cb in range(H // 16):
            rv0[r, pl.ds(cb * 16, 16)] = jnp.zeros((16,), jnp.float32)

    rbase = s * RPT
    for q in range(RPT // CH):
        pltpu.sync_copy(rv0, acc.at[pl.ds(rbase + q * CH, CH)])
    plsc.subcore_barrier()

    wbase = wid * EPP

    def issue_idx(j, q):
        pltpu.async_copy(sp_hbm.at[pl.ds(wbase + j * CH, CH)], ISRC[q], ES[q])
        pltpu.async_copy(dp_hbm.at[pl.ds(wbase + j * CH, CH)], IDST[q], ES[q])

    def wait_idx(j, q):
        pltpu.make_async_copy(sp_hbm.at[pl.ds(wbase + j * CH, CH)], ISRC[q], ES[q]).wait()
        pltpu.make_async_copy(dp_hbm.at[pl.ds(wbase + j * CH, CH)], IDST[q], ES[q]).wait()

    def issue(j, b, q):
        pltpu.async_copy(xc_hbm.at[ISRC[q]], RV[b], GS[b])
        pltpu.async_copy(w_hbm.at[pl.ds(wbase + j * CH, CH)], WV[b], WS[b])

    def wait_gw(j, b, q):
        pltpu.make_async_copy(xc_hbm.at[ISRC[q]], RV[b], GS[b]).wait()
        pltpu.make_async_copy(w_hbm.at[pl.ds(wbase + j * CH, CH)], WV[b], WS[b]).wait()

    def mult(b):
        @pl.loop(0, CH)
        def _(r):
            for cb in range(H // 16):
                sl = pl.ds(cb * 16, 16)
                RV[b][r, sl] = RV[b][r, sl] * WV[b][r, sl]

    def scat(j, b, q):
        pltpu.async_copy(RV[b], acc.at[IDST[q]], SS[b], add=True)

    def wait_scat(j, b, q):
        pltpu.make_async_copy(RV[b], acc.at[IDST[q]], SS[b]).wait()

    for q in range(4):
        issue_idx(q, q)
    wait_idx(0, 0)
    issue(0, 0, 0)
    wait_idx(1, 1)
    issue(1, 1, 1)

    @pl.loop(0, (NF2 - 2) // 4)
    def _(jj):
        for u in range(4):
            j = jj * 4 + u
            b = u % 2
            q = u
            wait_gw(j, b, q)
            mult(b)
            scat(j, b, q)
            wait_scat(j, b, q)

            @pl.when(j + 4 < NF2)
            def _():
                issue_idx(j + 4, q)

            wait_idx(j + 2, (q + 2) % 4)
            issue(j + 2, b, (q + 2) % 4)

    for b in range(2):
        j = NF2 - 2 + b
        wait_gw(j, b, j % 4)
        mult(b)
        scat(j, b, j % 4)
    wait_scat(NF2 - 2, 0, (NF2 - 2) % 4)
    wait_scat(NF2 - 1, 1, (NF2 - 1) % 4)

    plsc.subcore_barrier()
    pltpu.sync_copy(acc.at[pl.ds(rbase, RPT)],
                    out_hbm.at[pl.ds(c * NPAD + rbase, RPT)])


@functools.lru_cache(maxsize=1)
def _sc_gms_built():
    mesh = plsc.VectorSubcoreMesh(core_axis_name="c", subcore_axis_name="s")
    return pl.kernel(
        _sc_body,
        mesh=mesh,
        out_type=jax.ShapeDtypeStruct((NC * NPAD, H), jnp.float32),
        scratch_types=[
            pltpu.VMEM_SHARED((NPAD, H), jnp.float32),
            pltpu.VMEM((CH, H), jnp.float32),
            pltpu.VMEM((CH, H), jnp.float32),
            pltpu.VMEM((CH, H), jnp.float32),
            pltpu.VMEM((CH, H), jnp.float32),
            pltpu.VMEM((CH,), jnp.int32),
            pltpu.VMEM((CH,), jnp.int32),
            pltpu.VMEM((CH,), jnp.int32),
            pltpu.VMEM((CH,), jnp.int32),
            pltpu.VMEM((CH,), jnp.int32),
            pltpu.VMEM((CH,), jnp.int32),
            pltpu.VMEM((CH,), jnp.int32),
            pltpu.VMEM((CH,), jnp.int32),
            pltpu.SemaphoreType.DMA,
            pltpu.SemaphoreType.DMA,
            pltpu.SemaphoreType.DMA,
            pltpu.SemaphoreType.DMA,
            pltpu.SemaphoreType.DMA,
            pltpu.SemaphoreType.DMA,
            pltpu.SemaphoreType.DMA,
            pltpu.SemaphoreType.DMA,
            pltpu.SemaphoreType.DMA,
            pltpu.SemaphoreType.DMA,
        ],
    )


def _sc_call(xc, w, srcp, dstp):
    return _sc_gms_built()(xc, w, srcp, dstp)


def kernel(z, pos, batch, edge_index, edge_weight, emb,
           mlp_w1, mlp_b1, mlp_w2, mlp_b2,
           conv_w1, conv_w2, conv_b2, int_w, int_b,
           lin1_w, lin1_b, lin2_w, lin2_b):
    src = edge_index[0].astype(jnp.int32)
    dst = edge_index[1].astype(jnp.int32)
    zp = jnp.pad(z.astype(jnp.int32), (0, NPAD - N)).reshape(NBLK, 1, RT_N)

    pad = ((0, 0), (0, EPP - EPW))
    ewp = jnp.pad(edge_weight.reshape(NW, EPW), pad).reshape(-1)
    srcp = jnp.pad(src.reshape(NW, EPW), pad).reshape(-1)
    dstp = jnp.pad(dst.reshape(NW, EPW), pad,
                   constant_values=NPAD - 1).reshape(-1)

    ws = [_filter_call(ewp, mlp_w1[i], mlp_b1[i], mlp_w2[i], mlp_b2[i])
          for i in range(L)]

    h, xc = _embed_call(zp, emb, conv_w1[0])
    for i in range(L):
        parts = _sc_call(xc, ws[i], srcp, dstp)
        w1n = conv_w1[i + 1] if i + 1 < L else conv_w1[0]
        h, xc = _node_call(parts, h, conv_w2[i], conv_b2[i], int_w[i], int_b[i], w1n)

    return _final_call(h, lin1_w, lin1_b, lin2_w.reshape(1, H // 2), lin2_b)


# speedup vs baseline: 4.3416x; 1.0134x over previous
import functools

import jax
import jax.numpy as jnp
from jax import lax
from jax.experimental import pallas as pl
from jax.experimental.pallas import tpu as pltpu
from jax.experimental.pallas import tpu_sc as plsc

N = 10000
NPAD = 10240
E = 320000
H = 128
F = 128
G = 50
L = 6
CUT = 10.0

NC = 2
NS = 16
NW = NC * NS
EPW = E // NW
CH = 64
NF2 = 158
EPP = NF2 * CH
EPAD2 = NW * EPP
RPT = NPAD // NS

RT_F = 4096
RT_N = 1280
NBLK = NPAD // RT_N

_LOG2 = 0.6931471805599453


def _ssp(x):
    return jnp.maximum(x, 0.0) + jnp.log(1.0 + jnp.exp(-jnp.abs(x))) - _LOG2


def _filter_body(ew_ref, w1_ref, b1_ref, w2_ref, b2_ref, out_ref):
    ew = ew_ref[:]
    step = CUT / (G - 1)
    off = lax.broadcasted_iota(jnp.int32, (RT_F, G), 1).astype(jnp.float32) * step
    coeff = -0.5 / step ** 2
    a = jnp.exp(coeff * (ew[:, None] - off) ** 2)
    t = _ssp(jnp.dot(a, w1_ref[:], preferred_element_type=jnp.float32) + b1_ref[:])
    w = jnp.dot(t, w2_ref[:], preferred_element_type=jnp.float32) + b2_ref[:]
    c = 0.5 * (jnp.cos(ew * (jnp.pi / CUT)) + 1.0)
    out_ref[:] = w * c[:, None]


def _filter_call(ew, w1, b1, w2, b2):
    return pl.pallas_call(
        _filter_body,
        grid=(EPAD2 // RT_F,),
        in_specs=[
            pl.BlockSpec((RT_F,), lambda i: (i,)),
            pl.BlockSpec((G, F), lambda i: (0, 0)),
            pl.BlockSpec((F,), lambda i: (0,)),
            pl.BlockSpec((F, F), lambda i: (0, 0)),
            pl.BlockSpec((F,), lambda i: (0,)),
        ],
        out_specs=pl.BlockSpec((RT_F, F), lambda i: (i, 0)),
        out_shape=jax.ShapeDtypeStruct((EPAD2, F), jnp.float32),
    )(ew, w1, b1, w2, b2)


def _embed_body(z_ref, emb_ref, w1_ref, h_ref, xc_ref):
    z = z_ref[0, 0].reshape(RT_N, 1)
    oh = (lax.broadcasted_iota(jnp.int32, (RT_N, 120), 1) == z).astype(jnp.float32)
    h = jnp.dot(oh, emb_ref[:], preferred_element_type=jnp.float32)
    h_ref[:] = h
    xc_ref[:] = jnp.dot(h, w1_ref[:], preferred_element_type=jnp.float32)


def _embed_call(zp, emb, w1):
    return pl.pallas_call(
        _embed_body,
        grid=(NBLK,),
        in_specs=[
            pl.BlockSpec((1, 1, RT_N), lambda i: (i, 0, 0)),
            pl.BlockSpec((120, H), lambda i: (0, 0)),
            pl.BlockSpec((H, F), lambda i: (0, 0)),
        ],
        out_specs=[
            pl.BlockSpec((RT_N, H), lambda i: (i, 0)),
            pl.BlockSpec((RT_N, F), lambda i: (i, 0)),
        ],
        out_shape=[
            jax.ShapeDtypeStruct((NPAD, H), jnp.float32),
            jax.ShapeDtypeStruct((NPAD, F), jnp.float32),
        ],
    )(zp, emb, w1)


def _node_body(p0_ref, p1_ref, h_ref, w2_ref, b2_ref, iw_ref, ib_ref, w1n_ref,
               hn_ref, xcn_ref):
    agg = p0_ref[:] + p1_ref[:]
    x2 = jnp.dot(agg, w2_ref[:], preferred_element_type=jnp.float32) + b2_ref[:]
    hi = jnp.dot(_ssp(x2), iw_ref[:], preferred_element_type=jnp.float32) + ib_ref[:]
    hn = h_ref[:] + hi
    hn_ref[:] = hn
    xcn_ref[:] = jnp.dot(hn, w1n_ref[:], preferred_element_type=jnp.float32)


def _node_call(parts, h, w2, b2, iw, ib, w1n):
    return pl.pallas_call(
        _node_body,
        grid=(NBLK,),
        in_specs=[
            pl.BlockSpec((RT_N, H), lambda i: (i, 0)),
            pl.BlockSpec((RT_N, H), lambda i: (i + NBLK, 0)),
            pl.BlockSpec((RT_N, H), lambda i: (i, 0)),
            pl.BlockSpec((F, H), lambda i: (0, 0)),
            pl.BlockSpec((H,), lambda i: (0,)),
            pl.BlockSpec((H, H), lambda i: (0, 0)),
            pl.BlockSpec((H,), lambda i: (0,)),
            pl.BlockSpec((H, F), lambda i: (0, 0)),
        ],
        out_specs=[
            pl.BlockSpec((RT_N, H), lambda i: (i, 0)),
            pl.BlockSpec((RT_N, F), lambda i: (i, 0)),
        ],
        out_shape=[
            jax.ShapeDtypeStruct((NPAD, H), jnp.float32),
            jax.ShapeDtypeStruct((NPAD, F), jnp.float32),
        ],
    )(parts, parts, h, w2, b2, iw, ib, w1n)


def _final_body(h_ref, w1_ref, b1_ref, w2_ref, b2_ref, out_ref):
    i = pl.program_id(0)
    t = _ssp(jnp.dot(h_ref[:], w1_ref[:], preferred_element_type=jnp.float32) + b1_ref[:])
    r = jnp.sum(t * w2_ref[:], axis=1, keepdims=True) + b2_ref[:]
    rows = lax.broadcasted_iota(jnp.int32, (RT_N, 1), 0) + i * RT_N
    s = jnp.sum(jnp.where(rows < N, r, 0.0)).reshape(1, 1)

    @pl.when(i == 0)
    def _():
        out_ref[:] = s

    @pl.when(i != 0)
    def _():
        out_ref[:] = out_ref[:] + s


def _final_call(h, w1, b1, w2v, b2):
    return pl.pallas_call(
        _final_body,
        grid=(NBLK,),
        in_specs=[
            pl.BlockSpec((RT_N, H), lambda i: (i, 0)),
            pl.BlockSpec((H, H // 2), lambda i: (0, 0)),
            pl.BlockSpec((H // 2,), lambda i: (0,)),
            pl.BlockSpec((1, H // 2), lambda i: (0, 0)),
            pl.BlockSpec((1,), lambda i: (0,)),
        ],
        out_specs=pl.BlockSpec((1, 1), lambda i: (0, 0)),
        out_shape=jax.ShapeDtypeStruct((1, 1), jnp.float32),
    )(h, w1, b1, w2v, b2)


def _sc_body(xc_hbm, w_hbm, sp_hbm, dp_hbm, out_hbm,
             acc,
             rv0, rv1, rv2, wv0, wv1,
             is0, is1, is2, is3, is4, is5,
             id0, id1, id2, id3, id4, id5,
             gs0, gs1, gs2, ws0, ws1, ss0, ss1, ss2,
             es0, es1, es2, es3, es4, es5):
    c = lax.axis_index("c")
    s = lax.axis_index("s")
    wid = c * NS + s
    RV = (rv0, rv1, rv2)
    WV = (wv0, wv1)
    ISRC = (is0, is1, is2, is3, is4, is5)
    IDST = (id0, id1, id2, id3, id4, id5)
    GS = (gs0, gs1, gs2)
    WS = (ws0, ws1)
    SS = (ss0, ss1, ss2)
    ES = (es0, es1, es2, es3, es4, es5)

    @pl.loop(0, CH)
    def _(r):
        for cb in range(H // 16):
            rv0[r, pl.ds(cb * 16, 16)] = jnp.zeros((16,), jnp.float32)

    rbase = s * RPT
    for q in range(RPT // CH):
        pltpu.sync_copy(rv0, acc.at[pl.ds(rbase + q * CH, CH)])
    plsc.subcore_barrier()

    wbase = wid * EPP

    def issue_idx(j, q):
        pltpu.async_copy(sp_hbm.at[pl.ds(wbase + j * CH, CH)], ISRC[q], ES[q])
        pltpu.async_copy(dp_hbm.at[pl.ds(wbase + j * CH, CH)], IDST[q], ES[q])

    def wait_idx(j, q):
        pltpu.make_async_copy(sp_hbm.at[pl.ds(wbase + j * CH, CH)], ISRC[q], ES[q]).wait()
        pltpu.make_async_copy(dp_hbm.at[pl.ds(wbase + j * CH, CH)], IDST[q], ES[q]).wait()

    def issue(j, g, w, q):
        pltpu.async_copy(xc_hbm.at[ISRC[q]], RV[g], GS[g])
        pltpu.async_copy(w_hbm.at[pl.ds(wbase + j * CH, CH)], WV[w], WS[w])

    def wait_gw(j, g, w, q):
        pltpu.make_async_copy(xc_hbm.at[ISRC[q]], RV[g], GS[g]).wait()
        pltpu.make_async_copy(w_hbm.at[pl.ds(wbase + j * CH, CH)], WV[w], WS[w]).wait()

    def mult(g, w):
        @plsc.parallel_loop(0, CH, unroll=2)
        def _(r):
            for cb in range(H // 16):
                sl = pl.ds(cb * 16, 16)
                RV[g][r, sl] = RV[g][r, sl] * WV[w][r, sl]

    def scat(j, g, q):
        pltpu.async_copy(RV[g], acc.at[IDST[q]], SS[g], add=True)

    def wait_scat(j, g, q):
        pltpu.make_async_copy(RV[g], acc.at[IDST[q]], SS[g]).wait()

    for q in range(5):
        issue_idx(q, q)
    wait_idx(0, 0)
    issue(0, 0, 0, 0)
    wait_idx(1, 1)
    issue(1, 1, 1, 1)

    @pl.loop(0, (NF2 - 2) // 6)
    def _(jj):
        for u in range(6):
            j = jj * 6 + u
            g = u % 3
            w = u % 2
            q = u
            wait_gw(j, g, w, q)
            mult(g, w)
            scat(j, g, q)

            @pl.when(j >= 1)
            def _():
                wait_scat(j - 1, (u + 2) % 3, (u + 5) % 6)

            @pl.when(j + 5 < NF2)
            def _():
                issue_idx(j + 5, (u + 5) % 6)

            wait_idx(j + 2, (u + 2) % 6)
            issue(j + 2, (u + 2) % 3, u % 2, (u + 2) % 6)

    wait_scat(NF2 - 3, (NF2 - 3) % 3, (NF2 - 3) % 6)
    for u in range(2):
        j = NF2 - 2 + u
        wait_gw(j, j % 3, j % 2, j % 6)
        mult(j % 3, j % 2)
        scat(j, j % 3, j % 6)
    wait_scat(NF2 - 2, (NF2 - 2) % 3, (NF2 - 2) % 6)
    wait_scat(NF2 - 1, (NF2 - 1) % 3, (NF2 - 1) % 6)

    plsc.subcore_barrier()
    pltpu.sync_copy(acc.at[pl.ds(rbase, RPT)],
                    out_hbm.at[pl.ds(c * NPAD + rbase, RPT)])


@functools.lru_cache(maxsize=1)
def _sc_gms_built():
    mesh = plsc.VectorSubcoreMesh(core_axis_name="c", subcore_axis_name="s")
    return pl.kernel(
        _sc_body,
        mesh=mesh,
        out_type=jax.ShapeDtypeStruct((NC * NPAD, H), jnp.float32),
        scratch_types=(
            [pltpu.VMEM_SHARED((NPAD, H), jnp.float32)]
            + [pltpu.VMEM((CH, H), jnp.float32)] * 5
            + [pltpu.VMEM((CH,), jnp.int32)] * 12
            + [pltpu.SemaphoreType.DMA] * 14
        ),
    )


def _sc_call(xc, w, srcp, dstp):
    return _sc_gms_built()(xc, w, srcp, dstp)


def kernel(z, pos, batch, edge_index, edge_weight, emb,
           mlp_w1, mlp_b1, mlp_w2, mlp_b2,
           conv_w1, conv_w2, conv_b2, int_w, int_b,
           lin1_w, lin1_b, lin2_w, lin2_b):
    src = edge_index[0].astype(jnp.int32)
    dst = edge_index[1].astype(jnp.int32)
    zp = jnp.pad(z.astype(jnp.int32), (0, NPAD - N)).reshape(NBLK, 1, RT_N)

    pad = ((0, 0), (0, EPP - EPW))
    ewp = jnp.pad(edge_weight.reshape(NW, EPW), pad).reshape(-1)
    srcp = jnp.pad(src.reshape(NW, EPW), pad).reshape(-1)
    dstp = jnp.pad(dst.reshape(NW, EPW), pad,
                   constant_values=NPAD - 1).reshape(-1)

    ws = [_filter_call(ewp, mlp_w1[i], mlp_b1[i], mlp_w2[i], mlp_b2[i])
          for i in range(L)]

    h, xc = _embed_call(zp, emb, conv_w1[0])
    for i in range(L):
        parts = _sc_call(xc, ws[i], srcp, dstp)
        w1n = conv_w1[i + 1] if i + 1 < L else conv_w1[0]
        h, xc = _node_call(parts, h, conv_w2[i], conv_b2[i], int_w[i], int_b[i], w1n)

    return _final_call(h, lin1_w, lin1_b, lin2_w.reshape(1, H // 2), lin2_b)
